# baseline (speedup 1.0000x reference)
import functools

import jax
import jax.numpy as jnp
from jax import lax
from jax.experimental import pallas as pl
from jax.experimental.pallas import tpu as pltpu
from jax.experimental.pallas import tpu_sc as plsc

_NC = 2
_NS = 16
_NW = _NC * _NS


def _sc_gather(table, idx):
    n = idx.shape[0]
    d = table.shape[1]
    bpw = n // _NW
    mesh = plsc.VectorSubcoreMesh(core_axis_name="c", subcore_axis_name="s")

    @functools.partial(
        pl.kernel,
        mesh=mesh,
        out_type=jax.ShapeDtypeStruct((n, d), jnp.float32),
        scratch_types=[
            pltpu.VMEM((bpw,), jnp.int32),
            pltpu.VMEM((bpw, d), jnp.float32),
            pltpu.SemaphoreType.DMA,
        ],
        compiler_params=pltpu.CompilerParams(use_tc_tiling_on_sc=False),
    )
    def k(table_hbm, idx_hbm, out_hbm, idx_v, rows_v, sem):
        wid = lax.axis_index("s") * _NC + lax.axis_index("c")
        base = wid * bpw
        pltpu.sync_copy(idx_hbm.at[pl.ds(base, bpw)], idx_v)
        pltpu.async_copy(table_hbm.at[idx_v], rows_v, sem).wait()
        pltpu.sync_copy(rows_v, out_hbm.at[pl.ds(base, bpw)])

    return k(table, idx)


def _mm_body(e_ref, w_ref, b_ref, o_ref):
    wt = jnp.transpose(w_ref[...])
    o_ref[...] = (
        lax.dot_general(
            e_ref[...], wt,
            (((1,), (0,)), ((), ())),
            preferred_element_type=jnp.float32,
        )
        + b_ref[...]
    )


def _tc_matmul(e, w, b, tv):
    bsz, k = e.shape
    v = w.shape[0]
    grid = pl.cdiv(v, tv)
    return pl.pallas_call(
        _mm_body,
        grid=(grid,),
        in_specs=[
            pl.BlockSpec((bsz, k), lambda j: (0, 0)),
            pl.BlockSpec((tv, k), lambda j: (j, 0)),
            pl.BlockSpec((1, tv), lambda j: (0, j)),
        ],
        out_specs=pl.BlockSpec((bsz, tv), lambda j: (0, j)),
        out_shape=jax.ShapeDtypeStruct((bsz, v), jnp.float32),
    )(e, w, b.reshape(1, v))


def kernel(x, embed_table, fc_w, fc_b):
    bsz, ngram = x.shape
    idx = x.reshape(-1).astype(jnp.int32)
    e = _sc_gather(embed_table, idx)
    e = e.reshape(bsz, ngram * embed_table.shape[1])
    return _tc_matmul(e, fc_w, fc_b, tv=4096)


# speedup vs baseline: 2.9932x; 2.9932x over previous
import functools

import jax
import jax.numpy as jnp
from jax import lax
from jax.experimental import pallas as pl
from jax.experimental.pallas import tpu as pltpu
from jax.experimental.pallas import tpu_sc as plsc

_NC = 2
_NS = 16
_NW = _NC * _NS


def _sc_gather(table, idx):
    n = idx.shape[0]
    d = table.shape[1]
    bpw = n // _NW
    mesh = plsc.VectorSubcoreMesh(core_axis_name="c", subcore_axis_name="s")

    @functools.partial(
        pl.kernel,
        mesh=mesh,
        out_type=jax.ShapeDtypeStruct((n, d), jnp.float32),
        scratch_types=[
            pltpu.VMEM((bpw,), jnp.int32),
            pltpu.VMEM((bpw, d), jnp.float32),
            pltpu.SemaphoreType.DMA,
        ],
        compiler_params=pltpu.CompilerParams(use_tc_tiling_on_sc=False),
    )
    def k(table_hbm, idx_hbm, out_hbm, idx_v, rows_v, sem):
        wid = lax.axis_index("s") * _NC + lax.axis_index("c")
        base = wid * bpw
        pltpu.sync_copy(idx_hbm.at[pl.ds(base, bpw)], idx_v)
        pltpu.async_copy(table_hbm.at[idx_v], rows_v, sem).wait()
        pltpu.sync_copy(rows_v, out_hbm.at[pl.ds(base, bpw)])

    return k(table, idx)


def _mm_body(e_ref, wt_ref, b_ref, o_ref):
    et = jnp.transpose(e_ref[...])
    wv = jnp.transpose(wt_ref[...])
    acc = lax.dot_general(
        wv, et, (((1,), (0,)), ((), ())),
        preferred_element_type=jnp.float32,
    )
    bias = jnp.transpose(b_ref[...])
    o_ref[...] = acc + bias


def _tc_matmul_t(e, wt, b, tv):
    bsz, k = e.shape
    v = wt.shape[1]
    grid = pl.cdiv(v, tv)
    return pl.pallas_call(
        _mm_body,
        grid=(grid,),
        in_specs=[
            pl.BlockSpec((bsz, k), lambda j: (0, 0)),
            pl.BlockSpec((k, tv), lambda j: (0, j)),
            pl.BlockSpec((1, tv), lambda j: (0, j)),
        ],
        out_specs=pl.BlockSpec((tv, bsz), lambda j: (j, 0)),
        out_shape=jax.ShapeDtypeStruct((v, bsz), jnp.float32),
    )(e, wt, b.reshape(1, v))


def kernel(x, embed_table, fc_w, fc_b):
    bsz, ngram = x.shape
    idx = x.reshape(-1).astype(jnp.int32)
    e = _sc_gather(embed_table, idx)
    e = e.reshape(bsz, ngram * embed_table.shape[1])
    lt = _tc_matmul_t(e, fc_w.T, fc_b, tv=2048)
    return lt.T


# speedup vs baseline: 3.0111x; 1.0060x over previous
import functools

import jax
import jax.numpy as jnp
from jax import lax
from jax.experimental import pallas as pl
from jax.experimental.pallas import tpu as pltpu
from jax.experimental.pallas import tpu_sc as plsc

_NC = 2
_NS = 16
_NW = _NC * _NS


def _sc_gather(table, idx):
    n = idx.shape[0]
    d = table.shape[1]
    bpw = n // _NW
    mesh = plsc.VectorSubcoreMesh(core_axis_name="c", subcore_axis_name="s")

    @functools.partial(
        pl.kernel,
        mesh=mesh,
        out_type=jax.ShapeDtypeStruct((n, d), jnp.float32),
        scratch_types=[
            pltpu.VMEM((bpw,), jnp.int32),
            pltpu.VMEM((bpw, d), jnp.float32),
            pltpu.SemaphoreType.DMA,
        ],
        compiler_params=pltpu.CompilerParams(use_tc_tiling_on_sc=False),
    )
    def k(table_hbm, idx_hbm, out_hbm, idx_v, rows_v, sem):
        wid = lax.axis_index("s") * _NC + lax.axis_index("c")
        base = wid * bpw
        pltpu.sync_copy(idx_hbm.at[pl.ds(base, bpw)], idx_v)
        pltpu.async_copy(table_hbm.at[idx_v], rows_v, sem).wait()
        pltpu.sync_copy(rows_v, out_hbm.at[pl.ds(base, bpw)])

    return k(table, idx)


def _mm_body(e_ref, wt_ref, b_ref, o_ref):
    et = jnp.transpose(e_ref[...])
    wv = jnp.transpose(wt_ref[...])
    acc = lax.dot_general(
        wv, et, (((1,), (0,)), ((), ())),
        preferred_element_type=jnp.float32,
    )
    bias = jnp.transpose(b_ref[...])
    o_ref[...] = acc + bias


def _tc_matmul_t(e, wt, b, tv):
    bsz, k = e.shape
    v = wt.shape[1]
    grid = pl.cdiv(v, tv)
    return pl.pallas_call(
        _mm_body,
        grid=(grid,),
        in_specs=[
            pl.BlockSpec((bsz, k), lambda j: (0, 0)),
            pl.BlockSpec((k, tv), lambda j: (0, j)),
            pl.BlockSpec((1, tv), lambda j: (0, j)),
        ],
        out_specs=pl.BlockSpec((tv, bsz), lambda j: (j, 0)),
        out_shape=jax.ShapeDtypeStruct((v, bsz), jnp.float32),
    )(e, wt, b.reshape(1, v))


def kernel(x, embed_table, fc_w, fc_b):
    bsz, ngram = x.shape
    idx = x.reshape(-1).astype(jnp.int32)
    e = _sc_gather(embed_table, idx)
    e = e.reshape(bsz, ngram * embed_table.shape[1])
    lt = _tc_matmul_t(e, fc_w.T, fc_b, tv=4096)
    return lt.T


# speedup vs baseline: 3.5689x; 1.1852x over previous
import functools

import jax
import jax.numpy as jnp
from jax import lax
from jax.experimental import pallas as pl
from jax.experimental.pallas import tpu as pltpu
from jax.experimental.pallas import tpu_sc as plsc

_NC = 2
_NS = 16
_NW = _NC * _NS


def _sc_gather(table, idx):
    n = idx.shape[0]
    d = table.shape[1]
    bpw = n // _NW
    mesh = plsc.VectorSubcoreMesh(core_axis_name="c", subcore_axis_name="s")

    @functools.partial(
        pl.kernel,
        mesh=mesh,
        out_type=jax.ShapeDtypeStruct((n, d), jnp.float32),
        scratch_types=[
            pltpu.VMEM((bpw,), jnp.int32),
            pltpu.VMEM((bpw, d), jnp.float32),
            pltpu.SemaphoreType.DMA,
        ],
        compiler_params=pltpu.CompilerParams(use_tc_tiling_on_sc=False),
    )
    def k(table_hbm, idx_hbm, out_hbm, idx_v, rows_v, sem):
        wid = lax.axis_index("s") * _NC + lax.axis_index("c")
        base = wid * bpw
        pltpu.sync_copy(idx_hbm.at[pl.ds(base, bpw)], idx_v)
        pltpu.async_copy(table_hbm.at[idx_v], rows_v, sem).wait()
        pltpu.sync_copy(rows_v, out_hbm.at[pl.ds(base, bpw)])

    return k(table, idx)


def _sc_gather_elem(flat_table, idx, v, d):
    n = idx.shape[0]
    bpw = n // _NW
    npw = bpw * d
    mesh = plsc.VectorSubcoreMesh(core_axis_name="c", subcore_axis_name="s")

    @functools.partial(
        pl.kernel,
        mesh=mesh,
        out_type=jax.ShapeDtypeStruct((n * d,), jnp.float32),
        scratch_types=[
            pltpu.VMEM((bpw,), jnp.int32),
            pltpu.VMEM((npw,), jnp.int32),
            pltpu.VMEM((npw,), jnp.float32),
            pltpu.SemaphoreType.DMA,
        ],
        compiler_params=pltpu.CompilerParams(use_tc_tiling_on_sc=False),
    )
    def k(tab_hbm, idx_hbm, out_hbm, tok_v, eidx_v, vals_v, sem):
        wid = lax.axis_index("s") * _NC + lax.axis_index("c")
        base = wid * bpw
        pltpu.sync_copy(idx_hbm.at[pl.ds(base, bpw)], tok_v)
        lanes = lax.iota(jnp.int32, 16)
        for g in range(bpw // 16):
            tv = tok_v[pl.ds(g * 16, 16)]
            for kk in range(d):
                eidx_v[pl.ds((g * d + kk) * 16, 16)] = tv + kk * v
        pltpu.async_copy(tab_hbm.at[eidx_v], vals_v, sem).wait()
        pltpu.sync_copy(vals_v, out_hbm.at[pl.ds(base * d, npw)])

    return k(flat_table, idx)


def _mm_body(e_ref, wt_ref, b_ref, o_ref):
    et = jnp.transpose(e_ref[...])
    wv = jnp.transpose(wt_ref[...])
    acc = lax.dot_general(
        wv, et, (((1,), (0,)), ((), ())),
        preferred_element_type=jnp.float32,
    )
    bias = jnp.transpose(b_ref[...])
    o_ref[...] = acc + bias


def _tc_matmul_t(e, wt, b, tv):
    bsz, k = e.shape
    v = wt.shape[1]
    grid = pl.cdiv(v, tv)
    return pl.pallas_call(
        _mm_body,
        grid=(grid,),
        in_specs=[
            pl.BlockSpec((bsz, k), lambda j: (0, 0)),
            pl.BlockSpec((k, tv), lambda j: (0, j)),
            pl.BlockSpec((1, tv), lambda j: (0, j)),
        ],
        out_specs=pl.BlockSpec((tv, bsz), lambda j: (j, 0)),
        out_shape=jax.ShapeDtypeStruct((v, bsz), jnp.float32),
    )(e, wt, b.reshape(1, v))


def kernel(x, embed_table, fc_w, fc_b):
    bsz, ngram = x.shape
    idx = x.reshape(-1).astype(jnp.int32)
    v, d = embed_table.shape
    n = bsz * ngram
    flat_table = embed_table.T.reshape(-1)
    raw = _sc_gather_elem(flat_table, idx, v, d)
    e = raw.reshape(_NW, n // (_NW * 16), d, 16).transpose(0, 1, 3, 2)
    e = e.reshape(bsz, ngram * d)
    lt = _tc_matmul_t(e, fc_w.T, fc_b, tv=2048)
    return lt.T


# speedup vs baseline: 3.6009x; 1.0090x over previous
import functools

import jax
import jax.numpy as jnp
from jax import lax
from jax.experimental import pallas as pl
from jax.experimental.pallas import tpu as pltpu
from jax.experimental.pallas import tpu_sc as plsc

_NC = 2
_NS = 16
_NW = _NC * _NS


def _sc_gather_elem(flat_table, idx, v, d):
    n = idx.shape[0]
    bpw = n // _NW
    npw = bpw * d
    mesh = plsc.VectorSubcoreMesh(core_axis_name="c", subcore_axis_name="s")

    @functools.partial(
        pl.kernel,
        mesh=mesh,
        out_type=jax.ShapeDtypeStruct((n * d,), jnp.float32),
        scratch_types=[
            pltpu.VMEM((bpw,), jnp.int32),
            pltpu.VMEM((npw,), jnp.int32),
            pltpu.VMEM((npw,), jnp.float32),
            pltpu.SemaphoreType.DMA,
        ],
        compiler_params=pltpu.CompilerParams(use_tc_tiling_on_sc=False),
    )
    def k(tab_hbm, idx_hbm, out_hbm, tok_v, eidx_v, vals_v, sem):
        wid = lax.axis_index("s") * _NC + lax.axis_index("c")
        base = wid * bpw
        pltpu.sync_copy(idx_hbm.at[pl.ds(base, bpw)], tok_v)
        lanes = lax.iota(jnp.int32, 16)
        for g in range(bpw // 16):
            tv = tok_v[pl.ds(g * 16, 16)]
            for kk in range(d):
                eidx_v[pl.ds((g * d + kk) * 16, 16)] = tv + kk * v
        pltpu.async_copy(tab_hbm.at[eidx_v], vals_v, sem).wait()
        pltpu.sync_copy(vals_v, out_hbm.at[pl.ds(base * d, npw)])

    return k(flat_table, idx)


def _mm_body(e_ref, wt_ref, b_ref, o_ref):
    et = jnp.transpose(e_ref[...])
    wv = jnp.transpose(wt_ref[...])
    acc = lax.dot_general(
        wv, et, (((1,), (0,)), ((), ())),
        preferred_element_type=jnp.float32,
    )
    bias = jnp.transpose(b_ref[...])
    o_ref[...] = acc + bias


def _tc_matmul_t(e, wt, b, tv):
    bsz, k = e.shape
    v = wt.shape[1]
    grid = pl.cdiv(v, tv)
    return pl.pallas_call(
        _mm_body,
        grid=(grid,),
        in_specs=[
            pl.BlockSpec((bsz, k), lambda j: (0, 0)),
            pl.BlockSpec((k, tv), lambda j: (0, j)),
            pl.BlockSpec((1, tv), lambda j: (0, j)),
        ],
        out_specs=pl.BlockSpec((tv, bsz), lambda j: (j, 0)),
        out_shape=jax.ShapeDtypeStruct((v, bsz), jnp.float32),
    )(e, wt, b.reshape(1, v))


def kernel(x, embed_table, fc_w, fc_b):
    bsz, ngram = x.shape
    idx = x.reshape(-1).astype(jnp.int32)
    v, d = embed_table.shape
    n = bsz * ngram
    flat_table = embed_table.T.reshape(-1)
    raw = _sc_gather_elem(flat_table, idx, v, d)
    e = raw.reshape(_NW, n // (_NW * 16), d, 16).transpose(0, 1, 3, 2)
    e = e.reshape(bsz, ngram * d)
    lt = _tc_matmul_t(e, fc_w.T, fc_b, tv=4096)
    return lt.T
